# Initial kernel scaffold; baseline (speedup 1.0000x reference)
import jax
import jax.numpy as jnp
from jax.experimental import pallas as pl


def kernel(entity_emb, relation_emb, head_index, relation_index, tail_index):
    raise NotImplementedError("write your pallas kernel here")


# speedup vs baseline: 1.2417x; 1.2417x over previous
import functools

import jax
import jax.numpy as jnp
from jax import lax
from jax.experimental import pallas as pl
from jax.experimental.pallas import tpu as pltpu
from jax.experimental.pallas import tpu_sc as plsc

NC = 2
NS = 16
L = 16
NW = NC * NS
B = 16384
D = 128
BPW = B // NW
C = 128
NCHUNK = BPW // C


def _sc_body(ent_hbm, rel_hbm, hidx_hbm, ridx_hbm, tidx_hbm, out_hbm,
             hidx_v, ridx_v, tidx_v, hbuf, rbuf, tbuf, red, outbuf, sem):
    wid = lax.axis_index("s") * NC + lax.axis_index("c")
    base_w = wid * BPW
    col = lax.iota(jnp.int32, L)

    def chunk_body(cidx, carry):
        base = base_w + cidx * C
        pltpu.sync_copy(hidx_hbm.at[pl.ds(base, C)], hidx_v)
        pltpu.sync_copy(ridx_hbm.at[pl.ds(base, C)], ridx_v)
        pltpu.sync_copy(tidx_hbm.at[pl.ds(base, C)], tidx_v)
        ch = pltpu.async_copy(ent_hbm.at[hidx_v], hbuf, sem)
        cr = pltpu.async_copy(rel_hbm.at[ridx_v], rbuf, sem)
        ct = pltpu.async_copy(ent_hbm.at[tidx_v], tbuf, sem)
        ch.wait()
        cr.wait()
        ct.wait()

        def group_body(g, gcarry):
            rbase = g * L
            scores = jnp.zeros((L,), jnp.float32)
            for i in range(L):
                row = rbase + i
                acc = None
                for jv in range(D // L):
                    s = pl.ds(jv * L, L)
                    p = hbuf[row, s] * rbuf[row, s] * tbuf[row, s]
                    acc = p if acc is None else acc + p
                scores = jnp.where(col == i, jnp.sum(acc), scores)
            outbuf[pl.ds(rbase, L)] = scores
            return gcarry

        lax.fori_loop(0, C // L, group_body, 0)
        pltpu.sync_copy(outbuf, out_hbm.at[pl.ds(base, C)])
        return carry

    lax.fori_loop(0, NCHUNK, chunk_body, 0)


_distmult_sc = functools.partial(
    pl.kernel,
    out_type=jax.ShapeDtypeStruct((B,), jnp.float32),
    mesh=plsc.VectorSubcoreMesh(
        core_axis_name="c", subcore_axis_name="s",
        num_cores=NC, num_subcores=NS),
    scratch_types=[
        pltpu.VMEM((C,), jnp.int32),
        pltpu.VMEM((C,), jnp.int32),
        pltpu.VMEM((C,), jnp.int32),
        pltpu.VMEM((C, D), jnp.float32),
        pltpu.VMEM((C, D), jnp.float32),
        pltpu.VMEM((C, D), jnp.float32),
        pltpu.VMEM((L * L,), jnp.float32),
        pltpu.VMEM((C,), jnp.float32),
        pltpu.SemaphoreType.DMA,
    ],
    compiler_params=pltpu.CompilerParams(needs_layout_passes=False),
)(_sc_body)


@jax.jit
def kernel(entity_emb, relation_emb, head_index, relation_index, tail_index):
    return _distmult_sc(
        entity_emb,
        relation_emb,
        head_index.astype(jnp.int32),
        relation_index.astype(jnp.int32),
        tail_index.astype(jnp.int32),
    )


# speedup vs baseline: 1.3323x; 1.0730x over previous
import functools

import jax
import jax.numpy as jnp
from jax import lax
from jax.experimental import pallas as pl
from jax.experimental.pallas import tpu as pltpu
from jax.experimental.pallas import tpu_sc as plsc

NC = 2
NS = 16
L = 16
NW = NC * NS
B = 16384
D = 128
BPW = B // NW
C = 128
NCHUNK = BPW // C


def _sc_body(ent_hbm, rel_hbm, hidx_hbm, ridx_hbm, tidx_hbm, out_hbm,
             hidx_v, ridx_v, tidx_v, hbufs, rbufs, tbufs, outbuf,
             sem0, sem1):
    wid = lax.axis_index("s") * NC + lax.axis_index("c")
    base_w = wid * BPW
    lane = lax.iota(jnp.int32, L)
    sems = [sem0, sem1]

    pltpu.sync_copy(hidx_hbm.at[pl.ds(base_w, BPW)], hidx_v)
    pltpu.sync_copy(ridx_hbm.at[pl.ds(base_w, BPW)], ridx_v)
    pltpu.sync_copy(tidx_hbm.at[pl.ds(base_w, BPW)], tidx_v)

    def fire(c, slot):
        s = pl.ds(c * C, C)
        pltpu.async_copy(ent_hbm.at[hidx_v.at[s]], hbufs.at[slot], sems[slot])
        pltpu.async_copy(rel_hbm.at[ridx_v.at[s]], rbufs.at[slot], sems[slot])
        pltpu.async_copy(ent_hbm.at[tidx_v.at[s]], tbufs.at[slot], sems[slot])

    def drain(slot):
        s = pl.ds(0, C)
        pltpu.make_async_copy(ent_hbm.at[hidx_v.at[s]], hbufs.at[slot],
                              sems[slot]).wait()
        pltpu.make_async_copy(rel_hbm.at[ridx_v.at[s]], rbufs.at[slot],
                              sems[slot]).wait()
        pltpu.make_async_copy(ent_hbm.at[tidx_v.at[s]], tbufs.at[slot],
                              sems[slot]).wait()

    def compute(c, slot):
        def group_body(g, gcarry):
            scores = jnp.zeros((L,), jnp.float32)
            for i in range(L):
                row = g * L + i
                acc = None
                for jv in range(D // L):
                    s = pl.ds(jv * L, L)
                    p = (hbufs[slot, row, s] * rbufs[slot, row, s]
                         * tbufs[slot, row, s])
                    acc = p if acc is None else acc + p
                scores = jnp.where(lane == i, jnp.sum(acc), scores)
            outbuf[pl.ds(c * C + g * L, L)] = scores
            return gcarry

        lax.fori_loop(0, C // L, group_body, 0)

    fire(0, 0)

    def pair_body(p, carry):
        c0 = p * 2
        fire(c0 + 1, 1)
        drain(0)
        compute(c0, 0)

        @pl.when(p + 1 < NCHUNK // 2)
        def _():
            fire(c0 + 2, 0)

        drain(1)
        compute(c0 + 1, 1)
        return carry

    lax.fori_loop(0, NCHUNK // 2, pair_body, 0)

    pltpu.sync_copy(outbuf, out_hbm.at[pl.ds(base_w, BPW)])


_distmult_sc = functools.partial(
    pl.kernel,
    out_type=jax.ShapeDtypeStruct((B,), jnp.float32),
    mesh=plsc.VectorSubcoreMesh(
        core_axis_name="c", subcore_axis_name="s",
        num_cores=NC, num_subcores=NS),
    scratch_types=[
        pltpu.VMEM((BPW,), jnp.int32),
        pltpu.VMEM((BPW,), jnp.int32),
        pltpu.VMEM((BPW,), jnp.int32),
        pltpu.VMEM((2, C, D), jnp.float32),
        pltpu.VMEM((2, C, D), jnp.float32),
        pltpu.VMEM((2, C, D), jnp.float32),
        pltpu.VMEM((BPW,), jnp.float32),
        pltpu.SemaphoreType.DMA,
        pltpu.SemaphoreType.DMA,
    ],
    compiler_params=pltpu.CompilerParams(needs_layout_passes=False),
)(_sc_body)


@jax.jit
def kernel(entity_emb, relation_emb, head_index, relation_index, tail_index):
    return _distmult_sc(
        entity_emb,
        relation_emb,
        head_index.astype(jnp.int32),
        relation_index.astype(jnp.int32),
        tail_index.astype(jnp.int32),
    )


# speedup vs baseline: 2.3279x; 1.7473x over previous
import functools

import jax
import jax.numpy as jnp
from jax import lax
from jax.experimental import pallas as pl
from jax.experimental.pallas import tpu as pltpu
from jax.experimental.pallas import tpu_sc as plsc

NC = 2
NS = 16
L = 16
NW = NC * NS
B = 16384
D = 128
BPW = B // NW
C = 128
NCHUNK = BPW // C


def _sc_body(ent_hbm, rel_hbm, hidx_hbm, ridx_hbm, tidx_hbm, out_hbm,
             hidx_v, ridx_v, tidx_v, hbufs, rbufs, tbufs, outbuf, red,
             sem0, sem1):
    wid = lax.axis_index("s") * NC + lax.axis_index("c")
    base_w = wid * BPW
    lane = lax.iota(jnp.int32, L)
    sems = [sem0, sem1]

    pltpu.sync_copy(hidx_hbm.at[pl.ds(base_w, BPW)], hidx_v)
    pltpu.sync_copy(ridx_hbm.at[pl.ds(base_w, BPW)], ridx_v)
    pltpu.sync_copy(tidx_hbm.at[pl.ds(base_w, BPW)], tidx_v)

    def fire(c, slot):
        s = pl.ds(c * C, C)
        pltpu.async_copy(ent_hbm.at[hidx_v.at[s]], hbufs.at[slot], sems[slot])
        pltpu.async_copy(rel_hbm.at[ridx_v.at[s]], rbufs.at[slot], sems[slot])
        pltpu.async_copy(ent_hbm.at[tidx_v.at[s]], tbufs.at[slot], sems[slot])

    def drain(slot):
        s = pl.ds(0, C)
        pltpu.make_async_copy(ent_hbm.at[hidx_v.at[s]], hbufs.at[slot],
                              sems[slot]).wait()
        pltpu.make_async_copy(rel_hbm.at[ridx_v.at[s]], rbufs.at[slot],
                              sems[slot]).wait()
        pltpu.make_async_copy(ent_hbm.at[tidx_v.at[s]], tbufs.at[slot],
                              sems[slot]).wait()

    def compute(c, slot):
        def group_body(g, gcarry):
            for i in range(L):
                row = g * L + i
                acc = None
                for jv in range(D // L):
                    s = pl.ds(jv * L, L)
                    p = (hbufs[slot, row, s] * rbufs[slot, row, s]
                         * tbufs[slot, row, s])
                    acc = p if acc is None else acc + p
                red[pl.ds(i * L, L)] = acc
            scores = None
            for l in range(L):
                v = plsc.load_gather(red, [lane * L + l])
                scores = v if scores is None else scores + v
            outbuf[pl.ds(c * C + g * L, L)] = scores
            return gcarry

        lax.fori_loop(0, C // L, group_body, 0)

    fire(0, 0)

    def pair_body(p, carry):
        c0 = p * 2
        fire(c0 + 1, 1)
        drain(0)
        compute(c0, 0)

        @pl.when(p + 1 < NCHUNK // 2)
        def _():
            fire(c0 + 2, 0)

        drain(1)
        compute(c0 + 1, 1)
        return carry

    lax.fori_loop(0, NCHUNK // 2, pair_body, 0)

    pltpu.sync_copy(outbuf, out_hbm.at[pl.ds(base_w, BPW)])


_distmult_sc = functools.partial(
    pl.kernel,
    out_type=jax.ShapeDtypeStruct((B,), jnp.float32),
    mesh=plsc.VectorSubcoreMesh(
        core_axis_name="c", subcore_axis_name="s",
        num_cores=NC, num_subcores=NS),
    scratch_types=[
        pltpu.VMEM((BPW,), jnp.int32),
        pltpu.VMEM((BPW,), jnp.int32),
        pltpu.VMEM((BPW,), jnp.int32),
        pltpu.VMEM((2, C, D), jnp.float32),
        pltpu.VMEM((2, C, D), jnp.float32),
        pltpu.VMEM((2, C, D), jnp.float32),
        pltpu.VMEM((BPW,), jnp.float32),
        pltpu.VMEM((L * L,), jnp.float32),
        pltpu.SemaphoreType.DMA,
        pltpu.SemaphoreType.DMA,
    ],
    compiler_params=pltpu.CompilerParams(needs_layout_passes=False),
)(_sc_body)


@jax.jit
def kernel(entity_emb, relation_emb, head_index, relation_index, tail_index):
    return _distmult_sc(
        entity_emb,
        relation_emb,
        head_index.astype(jnp.int32),
        relation_index.astype(jnp.int32),
        tail_index.astype(jnp.int32),
    )
